# baseline (speedup 1.0000x reference)
import functools

import jax
import jax.numpy as jnp
from jax import lax
from jax.experimental import pallas as pl
from jax.experimental.pallas import tpu as pltpu
from jax.experimental.pallas import tpu_sc as plsc

NC, NS = 2, 16
NW = NC * NS
BS = 16
IDXC = 128


def _bdot(a, b):
    return jnp.dot(a.astype(jnp.bfloat16), b.astype(jnp.bfloat16),
                   preferred_element_type=jnp.float32)


def _sc_gather_means(xf, idxn, idx1, idx2, tb, s1, s2):
    d = xf.shape[1]
    nblk = tb // (NW * BS)
    n1 = BS * s1
    n2 = BS * s1 * s2
    mesh = plsc.VectorSubcoreMesh(core_axis_name="c", subcore_axis_name="s",
                                  num_cores=NC, num_subcores=NS)
    c2 = [(o, min(IDXC, n2 - o)) for o in range(0, n2, IDXC)]

    @functools.partial(
        pl.kernel,
        out_type=[jax.ShapeDtypeStruct((tb, d), jnp.float32),
                  jax.ShapeDtypeStruct((tb * s1, d), jnp.float32),
                  jax.ShapeDtypeStruct((tb, d), jnp.float32),
                  jax.ShapeDtypeStruct((tb * s1, d), jnp.float32)],
        mesh=mesh,
        compiler_params=pltpu.CompilerParams(use_tc_tiling_on_sc=False),
        scratch_types=[
            pltpu.VMEM((nblk, BS), jnp.int32),
            pltpu.VMEM((nblk, n1), jnp.int32),
            pltpu.VMEM((nblk, n2), jnp.int32),
            [pltpu.VMEM((BS, d), jnp.float32) for _ in range(2)],
            [pltpu.VMEM((n1, d), jnp.float32) for _ in range(2)],
            [pltpu.VMEM((n2, d), jnp.float32) for _ in range(2)],
            [pltpu.VMEM((BS, d), jnp.float32) for _ in range(2)],
            [pltpu.VMEM((n1, d), jnp.float32) for _ in range(2)],
            [pltpu.SemaphoreType.DMA for _ in range(2)],
            [pltpu.SemaphoreType.DMA for _ in range(2)],
            pltpu.SemaphoreType.DMA,
        ],
    )
    def k(xf_hbm, idxn_hbm, idx1_hbm, idx2_hbm,
          xsel_hbm, x1_hbm, na_hbm, nb_hbm,
          idxn_v, idx1_v, idx2_v, xsel_vs, xr1_vs, xr2_vs, na_vs, nb_vs,
          sems, wsems, isem):
        wid = lax.axis_index("s") * NC + lax.axis_index("c")
        pltpu.async_copy(idxn_hbm.at[wid], idxn_v, isem).wait()
        pltpu.async_copy(idx1_hbm.at[wid], idx1_v, isem).wait()
        pltpu.async_copy(idx2_hbm.at[wid], idx2_v, isem).wait()

        def gather_cps(fn, gc, p):
            cps = [fn(xf_hbm.at[idxn_v.at[gc]], xsel_vs[p], sems[p]),
                   fn(xf_hbm.at[idx1_v.at[gc]], xr1_vs[p], sems[p])]
            for o, c in c2:
                cps.append(fn(xf_hbm.at[idx2_v.at[gc, pl.ds(o, c)]],
                              xr2_vs[p].at[pl.ds(o, c)], sems[p]))
            return cps

        def gather_waits(p):
            cps = [plan(xf_hbm.at[pl.ds(0, BS)], xsel_vs[p], sems[p]),
                   plan(xf_hbm.at[pl.ds(0, n1)], xr1_vs[p], sems[p])]
            for o, c in c2:
                cps.append(plan(xf_hbm.at[pl.ds(0, c)],
                                xr2_vs[p].at[pl.ds(o, c)], sems[p]))
            for cp in cps:
                cp.wait()

        def write_cps(fn, gc, p):
            base = (wid * nblk + gc) * BS
            return [fn(xsel_vs[p], xsel_hbm.at[pl.ds(base, BS)], wsems[p]),
                    fn(xr1_vs[p], x1_hbm.at[pl.ds(base * s1, n1)], wsems[p]),
                    fn(na_vs[p], na_hbm.at[pl.ds(base, BS)], wsems[p]),
                    fn(nb_vs[p], nb_hbm.at[pl.ds(base * s1, n1)], wsems[p])]

        def wait_all(cps):
            for cp in cps:
                cp.wait()

        issue = pltpu.async_copy
        plan = pltpu.make_async_copy

        def compute(gc, p):
            xr1_v = xr1_vs[p]
            xr2_v = xr2_vs[p]
            na_v = na_vs[p]
            nb_v = nb_vs[p]

            def seed(si, c3):
                b1 = si * s1
                b2 = si * s1 * s2
                for kk in range(d // 16):
                    col = pl.ds(kk * 16, 16)
                    qa = xr1_v[b1, col]
                    for j in range(1, s1):
                        qa = qa + xr1_v[b1 + j, col]
                    na_v[si, col] = qa / float(s1)
                    for j in range(s1):
                        qb = xr2_v[b2 + 2 * j, col]
                        for jj in range(1, s2):
                            qb = qb + xr2_v[b2 + 2 * j + jj, col]
                        nb_v[b1 + j, col] = qb / float(s2)
                return c3

            lax.fori_loop(0, BS, seed, 0, unroll=False)

        gather_cps(issue, 0, 0)

        def phase(gc, p):
            @pl.when(gc >= 1)
            def _():
                wait_all(write_cps(plan, gc - 1, 1 - p))

            @pl.when(gc + 1 < nblk)
            def _():
                gather_cps(issue, gc + 1, 1 - p)

            gather_waits(p)
            compute(gc, p)
            write_cps(issue, gc, p)

        def pair(g2, carry):
            for p in range(2):
                phase(g2 * 2 + p, p)
            return carry

        lax.fori_loop(0, nblk // 2, pair, 0, unroll=False)
        if nblk % 2:
            phase(nblk - 1, 0)
        wait_all(write_cps(plan, nblk - 1, (nblk - 1) % 2))

    return k(xf, idxn, idx1, idx2)


def _tc_fused(xsel3, x13, na3, nb3, Wl0, bl0, Wr0, br0,
              w1cat, b1cat, wp, bp_row, s1, mt=256):
    t_steps, bpad, d = xsel3.shape
    h1 = Wl0.shape[1]
    h2 = w1cat.shape[1]
    out_f = wp.shape[1]
    mb = mt * s1

    def body(xsel_ref, x1_ref, na_ref, nb_ref, wl0_ref, bl0_ref, wr0_ref,
             br0_ref, w1_ref, b1_ref, wp_ref, bp_ref, out_ref):
        wl0 = wl0_ref[...]
        wr0 = wr0_ref[...]
        bias0 = bl0_ref[0][None, :]
        bias0b = br0_ref[0][None, :]
        rows = lax.broadcasted_iota(jnp.int32, (mt, mb), 0)
        cols = lax.broadcasted_iota(jnp.int32, (mt, mb), 1)
        gsum = (cols // s1 == rows).astype(jnp.bfloat16)
        spikes = []
        for t in range(t_steps):
            pre_a = _bdot(xsel_ref[t], wl0) + bias0 + _bdot(na_ref[t], wr0) + bias0b
            a = (pre_a >= 1.0).astype(jnp.float32)
            pre_b = _bdot(x1_ref[t], wl0) + bias0 + _bdot(nb_ref[t], wr0) + bias0b
            b = (pre_b >= 1.0).astype(jnp.bfloat16)
            bmean = jnp.dot(gsum, b, preferred_element_type=jnp.float32) / float(s1)
            ab = jnp.concatenate([a, bmean], axis=1)
            pre1 = _bdot(ab, w1_ref[...]) + b1_ref[0][None, :]
            u = pre1 - 1.0
            fwd = (u >= 0.0).astype(jnp.float32)
            g = jnp.where(u < -1.0, 0.0,
                          jnp.where(u > 1.0, 1.0,
                                    0.5 + u - jnp.sign(u) * u * u * 0.5))
            spikes.append(fwd - g + g)
        sp = jnp.concatenate(spikes, axis=1)
        out_ref[...] = _bdot(sp, wp_ref[...]) + bp_ref[0][None, :]

    return pl.pallas_call(
        body,
        grid=(bpad // mt,),
        in_specs=[pl.BlockSpec((t_steps, mt, d), lambda i: (0, i, 0)),
                  pl.BlockSpec((t_steps, mb, d), lambda i: (0, i, 0)),
                  pl.BlockSpec((t_steps, mt, d), lambda i: (0, i, 0)),
                  pl.BlockSpec((t_steps, mb, d), lambda i: (0, i, 0)),
                  pl.BlockSpec(Wl0.shape, lambda i: (0, 0)),
                  pl.BlockSpec((1, h1), lambda i: (0, 0)),
                  pl.BlockSpec(Wr0.shape, lambda i: (0, 0)),
                  pl.BlockSpec((1, h1), lambda i: (0, 0)),
                  pl.BlockSpec(w1cat.shape, lambda i: (0, 0)),
                  pl.BlockSpec((1, h2), lambda i: (0, 0)),
                  pl.BlockSpec(wp.shape, lambda i: (0, 0)),
                  pl.BlockSpec((1, out_f), lambda i: (0, 0))],
        out_specs=pl.BlockSpec((mt, out_f), lambda i: (i, 0)),
        out_shape=jax.ShapeDtypeStruct((bpad, out_f), jnp.float32),
    )(xsel3, x13, na3, nb3, Wl0, bl0.reshape(1, h1), Wr0, br0.reshape(1, h1),
      w1cat, b1cat, wp, bp_row)


def kernel(x, nodes, nbr1, nbr2, Wl0, bl0, Wr0, br0, Wl1, bl1, Wr1, br1, Wp, bp):
    t_steps, n, d = x.shape
    b = nodes.shape[0]
    s1 = nbr1.shape[1] // b
    s2 = nbr2.shape[1] // (b * s1)
    h2 = Wl1.shape[1]
    out_f = Wp.shape[1]

    nsplit = 10
    chunk = NW * BS * nsplit
    bpad = ((b + chunk - 1) // chunk) * chunk
    pad = bpad - b
    bph = bpad // nsplit

    toff = jnp.arange(t_steps, dtype=jnp.int32) * n
    nodes_p = jnp.pad(nodes, (0, pad))
    nbr1_p = jnp.pad(nbr1.reshape(t_steps, b, s1), ((0, 0), (0, pad), (0, 0)))
    nbr2_p = jnp.pad(nbr2.reshape(t_steps, b, s1 * s2), ((0, 0), (0, pad), (0, 0)))
    tb = t_steps * bph
    nblk = tb // (NW * BS)
    w1cat = jnp.concatenate([Wl1, Wr1], axis=0)
    b1cat = (bl1 + br1).reshape(1, h2)
    xf = x.reshape(t_steps * n, d)

    outs = []
    for h in range(nsplit):
        seeds = slice(h * bph, (h + 1) * bph)
        idxn = (nodes_p[None, seeds] + toff[:, None]).reshape(NW, nblk, BS)
        idx1 = (nbr1_p[:, seeds] + toff[:, None, None]).reshape(NW, nblk, BS * s1)
        idx2 = (nbr2_p[:, seeds] + toff[:, None, None]).reshape(NW, nblk, BS * s1 * s2)

        xsel, x1, na, nb = _sc_gather_means(xf, idxn, idx1, idx2, tb, s1, s2)

        xsel3 = xsel.reshape(t_steps, bph, d)
        x13 = x1.reshape(t_steps, bph * s1, d)
        na3 = na.reshape(t_steps, bph, d)
        nb3 = nb.reshape(t_steps, bph * s1, d)
        outs.append(_tc_fused(xsel3, x13, na3, nb3, Wl0, bl0, Wr0, br0,
                              w1cat, b1cat, Wp, bp.reshape(1, out_f), s1))
    return jnp.concatenate(outs, axis=0)[:b]


# speedup vs baseline: 1.0613x; 1.0613x over previous
import functools

import jax
import jax.numpy as jnp
from jax import lax
from jax.experimental import pallas as pl
from jax.experimental.pallas import tpu as pltpu
from jax.experimental.pallas import tpu_sc as plsc

NC, NS = 2, 16
NW = NC * NS
BS = 16
IDXC = 128


def _bdot(a, b):
    return jnp.dot(a.astype(jnp.bfloat16), b.astype(jnp.bfloat16),
                   preferred_element_type=jnp.float32)


def _sc_gather_means(xf, idxn, idx1, idx2, tb, s1, s2):
    d = xf.shape[1]
    nblk = tb // (NW * BS)
    n1 = BS * s1
    n2 = BS * s1 * s2
    mesh = plsc.VectorSubcoreMesh(core_axis_name="c", subcore_axis_name="s",
                                  num_cores=NC, num_subcores=NS)
    c2 = [(o, min(IDXC, n2 - o)) for o in range(0, n2, IDXC)]

    @functools.partial(
        pl.kernel,
        out_type=[jax.ShapeDtypeStruct((tb, d), jnp.float32),
                  jax.ShapeDtypeStruct((tb * s1, d), jnp.float32),
                  jax.ShapeDtypeStruct((tb, d), jnp.float32),
                  jax.ShapeDtypeStruct((tb * s1, d), jnp.float32)],
        mesh=mesh,
        compiler_params=pltpu.CompilerParams(use_tc_tiling_on_sc=False),
        scratch_types=[
            pltpu.VMEM((nblk, BS), jnp.int32),
            pltpu.VMEM((nblk, n1), jnp.int32),
            pltpu.VMEM((nblk, n2), jnp.int32),
            [pltpu.VMEM((BS, d), jnp.float32) for _ in range(2)],
            [pltpu.VMEM((n1, d), jnp.float32) for _ in range(2)],
            [pltpu.VMEM((n2, d), jnp.float32) for _ in range(2)],
            [pltpu.VMEM((BS, d), jnp.float32) for _ in range(2)],
            [pltpu.VMEM((n1, d), jnp.float32) for _ in range(2)],
            [pltpu.SemaphoreType.DMA for _ in range(2)],
            [pltpu.SemaphoreType.DMA for _ in range(2)],
            pltpu.SemaphoreType.DMA,
        ],
    )
    def k(xf_hbm, idxn_hbm, idx1_hbm, idx2_hbm,
          xsel_hbm, x1_hbm, na_hbm, nb_hbm,
          idxn_v, idx1_v, idx2_v, xsel_vs, xr1_vs, xr2_vs, na_vs, nb_vs,
          sems, wsems, isem):
        wid = lax.axis_index("s") * NC + lax.axis_index("c")
        pltpu.async_copy(idxn_hbm.at[wid], idxn_v, isem).wait()
        pltpu.async_copy(idx1_hbm.at[wid], idx1_v, isem).wait()
        pltpu.async_copy(idx2_hbm.at[wid], idx2_v, isem).wait()

        def gather_cps(fn, gc, p):
            cps = [fn(xf_hbm.at[idxn_v.at[gc]], xsel_vs[p], sems[p]),
                   fn(xf_hbm.at[idx1_v.at[gc]], xr1_vs[p], sems[p])]
            for o, c in c2:
                cps.append(fn(xf_hbm.at[idx2_v.at[gc, pl.ds(o, c)]],
                              xr2_vs[p].at[pl.ds(o, c)], sems[p]))
            return cps

        def gather_waits(p):
            cps = [plan(xf_hbm.at[pl.ds(0, BS)], xsel_vs[p], sems[p]),
                   plan(xf_hbm.at[pl.ds(0, n1)], xr1_vs[p], sems[p])]
            for o, c in c2:
                cps.append(plan(xf_hbm.at[pl.ds(0, c)],
                                xr2_vs[p].at[pl.ds(o, c)], sems[p]))
            for cp in cps:
                cp.wait()

        def write_cps(fn, gc, p):
            base = (wid * nblk + gc) * BS
            return [fn(xsel_vs[p], xsel_hbm.at[pl.ds(base, BS)], wsems[p]),
                    fn(xr1_vs[p], x1_hbm.at[pl.ds(base * s1, n1)], wsems[p]),
                    fn(na_vs[p], na_hbm.at[pl.ds(base, BS)], wsems[p]),
                    fn(nb_vs[p], nb_hbm.at[pl.ds(base * s1, n1)], wsems[p])]

        def wait_all(cps):
            for cp in cps:
                cp.wait()

        issue = pltpu.async_copy
        plan = pltpu.make_async_copy

        def compute(gc, p):
            xr1_v = xr1_vs[p]
            xr2_v = xr2_vs[p]
            na_v = na_vs[p]
            nb_v = nb_vs[p]

            def seed(si, c3):
                b1 = si * s1
                b2 = si * s1 * s2
                for kk in range(d // 16):
                    col = pl.ds(kk * 16, 16)
                    qa = xr1_v[b1, col]
                    for j in range(1, s1):
                        qa = qa + xr1_v[b1 + j, col]
                    na_v[si, col] = qa / float(s1)
                    for j in range(s1):
                        qb = xr2_v[b2 + 2 * j, col]
                        for jj in range(1, s2):
                            qb = qb + xr2_v[b2 + 2 * j + jj, col]
                        nb_v[b1 + j, col] = qb / float(s2)
                return c3

            lax.fori_loop(0, BS, seed, 0, unroll=False)

        gather_cps(issue, 0, 0)

        def phase(gc, p):
            @pl.when(gc >= 1)
            def _():
                wait_all(write_cps(plan, gc - 1, 1 - p))

            @pl.when(gc + 1 < nblk)
            def _():
                gather_cps(issue, gc + 1, 1 - p)

            gather_waits(p)
            compute(gc, p)
            write_cps(issue, gc, p)

        def pair(g2, carry):
            for p in range(2):
                phase(g2 * 2 + p, p)
            return carry

        lax.fori_loop(0, nblk // 2, pair, 0, unroll=False)
        if nblk % 2:
            phase(nblk - 1, 0)
        wait_all(write_cps(plan, nblk - 1, (nblk - 1) % 2))

    return k(xf, idxn, idx1, idx2)


def _tc_fused(xsel3, x13, na3, nb3, Wl0, bl0, Wr0, br0,
              w1cat, b1cat, wp, bp_row, s1, mt=256):
    t_steps, bpad, d = xsel3.shape
    h1 = Wl0.shape[1]
    h2 = w1cat.shape[1]
    out_f = wp.shape[1]
    mb = mt * s1

    def body(xsel_ref, x1_ref, na_ref, nb_ref, wl0_ref, bl0_ref, wr0_ref,
             br0_ref, w1_ref, b1_ref, wp_ref, bp_ref, out_ref):
        wl0 = wl0_ref[...]
        wr0 = wr0_ref[...]
        bias0 = bl0_ref[0][None, :]
        bias0b = br0_ref[0][None, :]
        rows = lax.broadcasted_iota(jnp.int32, (mt, mb), 0)
        cols = lax.broadcasted_iota(jnp.int32, (mt, mb), 1)
        gsum = (cols // s1 == rows).astype(jnp.bfloat16)
        spikes = []
        for t in range(t_steps):
            pre_a = _bdot(xsel_ref[t], wl0) + bias0 + _bdot(na_ref[t], wr0) + bias0b
            a = (pre_a >= 1.0).astype(jnp.float32)
            pre_b = _bdot(x1_ref[t], wl0) + bias0 + _bdot(nb_ref[t], wr0) + bias0b
            b = (pre_b >= 1.0).astype(jnp.bfloat16)
            bmean = jnp.dot(gsum, b, preferred_element_type=jnp.float32) / float(s1)
            ab = jnp.concatenate([a, bmean], axis=1)
            pre1 = _bdot(ab, w1_ref[...]) + b1_ref[0][None, :]
            u = pre1 - 1.0
            fwd = (u >= 0.0).astype(jnp.float32)
            g = jnp.where(u < -1.0, 0.0,
                          jnp.where(u > 1.0, 1.0,
                                    0.5 + u - jnp.sign(u) * u * u * 0.5))
            spikes.append(fwd - g + g)
        sp = jnp.concatenate(spikes, axis=1)
        out_ref[...] = _bdot(sp, wp_ref[...]) + bp_ref[0][None, :]

    return pl.pallas_call(
        body,
        grid=(bpad // mt,),
        in_specs=[pl.BlockSpec((t_steps, mt, d), lambda i: (0, i, 0)),
                  pl.BlockSpec((t_steps, mb, d), lambda i: (0, i, 0)),
                  pl.BlockSpec((t_steps, mt, d), lambda i: (0, i, 0)),
                  pl.BlockSpec((t_steps, mb, d), lambda i: (0, i, 0)),
                  pl.BlockSpec(Wl0.shape, lambda i: (0, 0)),
                  pl.BlockSpec((1, h1), lambda i: (0, 0)),
                  pl.BlockSpec(Wr0.shape, lambda i: (0, 0)),
                  pl.BlockSpec((1, h1), lambda i: (0, 0)),
                  pl.BlockSpec(w1cat.shape, lambda i: (0, 0)),
                  pl.BlockSpec((1, h2), lambda i: (0, 0)),
                  pl.BlockSpec(wp.shape, lambda i: (0, 0)),
                  pl.BlockSpec((1, out_f), lambda i: (0, 0))],
        out_specs=pl.BlockSpec((mt, out_f), lambda i: (i, 0)),
        out_shape=jax.ShapeDtypeStruct((bpad, out_f), jnp.float32),
    )(xsel3, x13, na3, nb3, Wl0, bl0.reshape(1, h1), Wr0, br0.reshape(1, h1),
      w1cat, b1cat, wp, bp_row)


def kernel(x, nodes, nbr1, nbr2, Wl0, bl0, Wr0, br0, Wl1, bl1, Wr1, br1, Wp, bp):
    t_steps, n, d = x.shape
    b = nodes.shape[0]
    s1 = nbr1.shape[1] // b
    s2 = nbr2.shape[1] // (b * s1)
    h2 = Wl1.shape[1]
    out_f = Wp.shape[1]

    nsplit = 5
    chunk = NW * BS * nsplit
    bpad = ((b + chunk - 1) // chunk) * chunk
    pad = bpad - b
    bph = bpad // nsplit

    toff = jnp.arange(t_steps, dtype=jnp.int32) * n
    nodes_p = jnp.pad(nodes, (0, pad))
    nbr1_p = jnp.pad(nbr1.reshape(t_steps, b, s1), ((0, 0), (0, pad), (0, 0)))
    nbr2_p = jnp.pad(nbr2.reshape(t_steps, b, s1 * s2), ((0, 0), (0, pad), (0, 0)))
    tb = t_steps * bph
    nblk = tb // (NW * BS)
    w1cat = jnp.concatenate([Wl1, Wr1], axis=0)
    b1cat = (bl1 + br1).reshape(1, h2)
    xf = x.reshape(t_steps * n, d)

    outs = []
    for h in range(nsplit):
        seeds = slice(h * bph, (h + 1) * bph)
        idxn = (nodes_p[None, seeds] + toff[:, None]).reshape(NW, nblk, BS)
        idx1 = (nbr1_p[:, seeds] + toff[:, None, None]).reshape(NW, nblk, BS * s1)
        idx2 = (nbr2_p[:, seeds] + toff[:, None, None]).reshape(NW, nblk, BS * s1 * s2)

        xsel, x1, na, nb = _sc_gather_means(xf, idxn, idx1, idx2, tb, s1, s2)

        xsel3 = xsel.reshape(t_steps, bph, d)
        x13 = x1.reshape(t_steps, bph * s1, d)
        na3 = na.reshape(t_steps, bph, d)
        nb3 = nb.reshape(t_steps, bph * s1, d)
        outs.append(_tc_fused(xsel3, x13, na3, nb3, Wl0, bl0, Wr0, br0,
                              w1cat, b1cat, Wp, bp.reshape(1, out_f), s1))
    return jnp.concatenate(outs, axis=0)[:b]
